# baseline (speedup 1.0000x reference)
import functools

import jax
import jax.numpy as jnp
from jax import lax
from jax.experimental import pallas as pl
from jax.experimental.pallas import tpu as pltpu
from jax.experimental.pallas import tpu_sc as plsc

N = 10000
E = 100000
XD = 200
H = 64
PAIRS = [(0, 0), (1, 1), (2, 2), (3, 3), (1, 0), (0, 1), (1, 2), (2, 1), (1, 3), (3, 1)]

NC, NS, L = 2, 16, 16
CH = 128
CHR = 4
EPS = E // NS
NCHUNK = 56
EPW = NCHUNK * CH
TRASH = N
NP = 10112
ZR = NP // NS

_P10_BY_DOM = ((0, 4), (1, 5, 7, 9), (2, 6), (3, 8))


def _make_scatter(k00, k01, k10, k11):
    kmax = max(k00, k01, k10, k11)
    mesh = plsc.VectorSubcoreMesh(core_axis_name="c", subcore_axis_name="s")

    @functools.partial(
        pl.kernel,
        mesh=mesh,
        out_type=jax.ShapeDtypeStruct((NC, 2, NP, H), jnp.float32),
        scratch_types=[
            pltpu.VMEM((NCHUNK * CH,), jnp.int32),
            pltpu.VMEM((NCHUNK * CH,), jnp.int32),
            pltpu.VMEM((CHR * CH, H), jnp.float32),
            pltpu.VMEM((CHR * CH, H), jnp.float32),
            pltpu.VMEM_SHARED((NP, H), jnp.float32),
            pltpu.SemaphoreType.DMA,
            pltpu.SemaphoreType.DMA,
        ],
        compiler_params=pltpu.CompilerParams(use_tc_tiling_on_sc=False,
                                             needs_layout_passes=False),
    )
    def scat(h_hbm, z_hbm, s00, d00, s01, d01, s10, d10, s11, d11, out,
             srcv, dstv, rb0, rb1, acc, sm0, sm1):
        cid = lax.axis_index("c")
        sid = lax.axis_index("s")
        stripe = pl.ds(sid * ZR, ZR)

        def rows(kk):
            return pl.ds(kk * CHR * CH, CHR * CH)

        def fire(kk, rb, sm):
            pltpu.async_copy(h_hbm.at[srcv.at[rows(kk)]], rb, sm)

        def wait(rb, sm):
            pltpu.make_async_copy(h_hbm.at[srcv.at[rows(0)]], rb, sm).wait()

        def scat_add(kk, rb):
            pltpu.sync_copy(rb, acc.at[dstv.at[rows(kk)]], add=True)

        def run(s_hbm, d_hbm, k):
            m = NCHUNK // CHR
            for s in range(k // NCHUNK):
                off = (sid * k + s * NCHUNK) * CH
                pltpu.sync_copy(s_hbm.at[pl.ds(off, NCHUNK * CH)], srcv)
                pltpu.sync_copy(d_hbm.at[pl.ds(off, NCHUNK * CH)], dstv)

                fire(0, rb0, sm0)
                fire(1, rb1, sm1)

                def body(i, c):
                    kk = 2 * i
                    wait(rb0, sm0)
                    scat_add(kk, rb0)
                    fire(kk + 2, rb0, sm0)
                    wait(rb1, sm1)
                    scat_add(kk + 1, rb1)
                    fire(kk + 3, rb1, sm1)
                    return c

                lax.fori_loop(0, m // 2 - 1, body, 0)
                wait(rb0, sm0)
                scat_add(m - 2, rb0)
                wait(rb1, sm1)
                scat_add(m - 1, rb1)

        def phase(t, s_hbm, d_hbm, k):
            pltpu.sync_copy(z_hbm.at[stripe], acc.at[stripe])
            plsc.subcore_barrier()
            run(s_hbm, d_hbm, k)
            plsc.subcore_barrier()
            pltpu.sync_copy(acc.at[stripe], out.at[cid, t, stripe])

        @pl.when(cid == 0)
        def _():
            phase(0, s00, d00, k00)
            phase(1, s01, d01, k01)

        @pl.when(cid == 1)
        def _():
            phase(0, s10, d10, k10)
            phase(1, s11, d11, k11)

    return scat


_scat4 = _make_scatter(NCHUNK, NCHUNK, NCHUNK, NCHUNK)
_scat10 = _make_scatter(*[len(p) * NCHUNK for p in _P10_BY_DOM])


CROWS = 2 * N + 16
mesh_a = plsc.VectorSubcoreMesh(core_axis_name="c", subcore_axis_name="s")


@functools.partial(
    pl.kernel,
    mesh=mesh_a,
    out_type=[
        jax.ShapeDtypeStruct((4 * NS * EPW,), jnp.float32),
        jax.ShapeDtypeStruct((4 * NS * EPW,), jnp.int32),
        jax.ShapeDtypeStruct((NC * NS * CROWS,), jnp.float32)
    ],
    scratch_types=[
        pltpu.VMEM((N,), jnp.float32),
        pltpu.VMEM((N,), jnp.float32),
        pltpu.VMEM((N + 16,), jnp.float32),
        pltpu.VMEM((N + 16,), jnp.float32),
        pltpu.VMEM((EPW,), jnp.int32),
        pltpu.VMEM((EPW,), jnp.int32),
        pltpu.VMEM((EPW,), jnp.float32),
        pltpu.VMEM((EPW,), jnp.float32),
        pltpu.VMEM((EPW,), jnp.float32),
        pltpu.VMEM((EPW,), jnp.int32),
        pltpu.VMEM((16,), jnp.int32),
        pltpu.VMEM((CROWS,), jnp.float32),
    ],
    compiler_params=pltpu.CompilerParams(use_tc_tiling_on_sc=False,
                                         needs_layout_passes=False),
)
def _attn(a0t, a1t, b0t, b1t, srcl, dstl, g0, g1, idx16, csrc, cdst,
          ws_o, de_o, cp_o,
          a0s, a1s, b0s, b1s, srcf, dstf, g0f, g1f, wsb, deb, idxv_v, cacc):
    cid = lax.axis_index("c")
    sid = lax.axis_index("s")
    pltpu.sync_copy(idx16, idxv_v)
    idxv = idxv_v[...]
    nv = jnp.full((16,), N, jnp.int32)

    def zero(i, c):
        cacc[pl.ds(i * 16, 16)] = jnp.zeros((16,), jnp.float32)
        return c

    lax.fori_loop(0, CROWS // 16, zero, 0)

    def wspair(p, with_c, coff):
        pltpu.sync_copy(a0t.at[pl.ds(p * N, N)], a0s)
        pltpu.sync_copy(a1t.at[pl.ds(p * N, N)], a1s)
        pltpu.sync_copy(b0t.at[pl.ds(p * N, N + 16)], b0s)
        pltpu.sync_copy(b1t.at[pl.ds(p * N, N + 16)], b1s)
        seg = pl.ds((p * NS + sid) * EPW, EPW)
        pltpu.sync_copy(srcl.at[seg], srcf)
        pltpu.sync_copy(dstl.at[seg], dstf)
        pltpu.sync_copy(g0.at[seg], g0f)
        pltpu.sync_copy(g1.at[seg], g1f)
        cofv = jnp.full((16,), coff, jnp.int32)

        def body(i, c):
            o = i * 16
            vs = srcf[pl.ds(o, 16)]
            vd = dstf[pl.ds(o, 16)]
            m0 = (plsc.load_gather(a0s, [vs]) + plsc.load_gather(b0s, [vd])) \
                + g0f[pl.ds(o, 16)]
            m1 = (plsc.load_gather(a1s, [vs]) + plsc.load_gather(b1s, [vd])) \
                + g1f[pl.ds(o, 16)]
            w = m1 > m0
            wsb[pl.ds(o, 16)] = jnp.where(w, 1.0, 0.0).astype(jnp.float32)
            deb[pl.ds(o, 16)] = jnp.where(w, vd, nv)
            if with_c:
                coeff = jnp.where(w & (vd == idxv), 1.0, 0.0).astype(jnp.float32)
                plsc.addupdate_scatter(cacc, [vs + cofv], coeff)
            return c

        lax.fori_loop(0, EPW // 16, body, 0)
        pltpu.sync_copy(wsb, ws_o.at[seg])
        pltpu.sync_copy(deb, de_o.at[seg])

    def cpair(k, coff):
        seg = pl.ds((k * NS + sid) * EPW, EPW)
        pltpu.sync_copy(csrc.at[seg], srcf)
        pltpu.sync_copy(cdst.at[seg], dstf)
        cofv = jnp.full((16,), coff, jnp.int32)

        def body(i, c):
            o = i * 16
            vs = srcf[pl.ds(o, 16)]
            vd = dstf[pl.ds(o, 16)]
            coeff = jnp.where(vd == idxv, 1.0, 0.0).astype(jnp.float32)
            plsc.addupdate_scatter(cacc, [vs + cofv], coeff)
            return c

        lax.fori_loop(0, EPW // 16, body, 0)

    @pl.when(cid == 0)
    def _():
        wspair(0, False, 0)
        wspair(1, True, N)
        cpair(0, 0)

    @pl.when(cid == 1)
    def _():
        wspair(2, False, 0)
        wspair(3, False, 0)
        cpair(1, 0)
        cpair(2, N)

    pltpu.sync_copy(cacc, cp_o.at[pl.ds((cid * NS + sid) * CROWS, CROWS)])


M4 = 4 * N
_BM = 2000


def _tc_proj(x, w, b):
    def body(x_ref, w_ref, b_ref, o_ref):
        o_ref[...] = jnp.maximum(
            jnp.dot(x_ref[...], w_ref[...], preferred_element_type=jnp.float32)
            + b_ref[...], 0.0)

    return pl.pallas_call(
        body,
        grid=(M4 // _BM,),
        in_specs=[pl.BlockSpec((_BM, XD), lambda i: (i, 0)),
                  pl.BlockSpec((XD, H), lambda i: (0, 0)),
                  pl.BlockSpec((1, H), lambda i: (0, 0))],
        out_specs=pl.BlockSpec((_BM, H), lambda i: (i, 0)),
        out_shape=jax.ShapeDtypeStruct((M4, H), jnp.float32),
    )(x, w, b)


def _tc_layer(h, m, ws, wm, b, w4=None):
    if w4 is None:
        def body(h_ref, m_ref, ws_ref, wm_ref, b_ref, o_ref):
            o_ref[...] = jnp.maximum(
                jnp.dot(h_ref[...], ws_ref[...], preferred_element_type=jnp.float32)
                + jnp.dot(m_ref[...], wm_ref[...], preferred_element_type=jnp.float32)
                + b_ref[...], 0.0)

        return pl.pallas_call(
            body,
            grid=(M4 // _BM,),
            in_specs=[pl.BlockSpec((_BM, H), lambda i: (i, 0)),
                      pl.BlockSpec((_BM, H), lambda i: (i, 0)),
                      pl.BlockSpec((H, H), lambda i: (0, 0)),
                      pl.BlockSpec((H, H), lambda i: (0, 0)),
                      pl.BlockSpec((1, H), lambda i: (0, 0))],
            out_specs=pl.BlockSpec((_BM, H), lambda i: (i, 0)),
            out_shape=jax.ShapeDtypeStruct((M4, H), jnp.float32),
        )(h, m, ws, wm, b)

    def body(h_ref, m_ref, ws_ref, wm_ref, b_ref, w4_ref, o_ref, t_ref):
        o = jnp.maximum(
            jnp.dot(h_ref[...], ws_ref[...], preferred_element_type=jnp.float32)
            + jnp.dot(m_ref[...], wm_ref[...], preferred_element_type=jnp.float32)
            + b_ref[...], 0.0)
        o_ref[...] = o
        t_ref[...] = jnp.dot(o, w4_ref[...], preferred_element_type=jnp.float32)

    return pl.pallas_call(
        body,
        grid=(M4 // _BM,),
        in_specs=[pl.BlockSpec((_BM, H), lambda i: (i, 0)),
                  pl.BlockSpec((_BM, H), lambda i: (i, 0)),
                  pl.BlockSpec((H, H), lambda i: (0, 0)),
                  pl.BlockSpec((H, H), lambda i: (0, 0)),
                  pl.BlockSpec((1, H), lambda i: (0, 0)),
                  pl.BlockSpec((H, 4), lambda i: (0, 0))],
        out_specs=[pl.BlockSpec((_BM, H), lambda i: (i, 0)),
                   pl.BlockSpec((_BM, 4), lambda i: (i, 0))],
        out_shape=[jax.ShapeDtypeStruct((M4, H), jnp.float32),
                   jax.ShapeDtypeStruct((M4, 4), jnp.float32)],
    )(h, m, ws, wm, b, w4)


_BR = 2000
_NG = M4 // _BR


def _tc_final(hc, cpart, hrow, ws2, wm2, b2, wcp, bcp):
    def body(hc_ref, cp_ref, hr_ref, ws2_ref, wm2_ref, b2_ref, wc_ref,
             bc_ref, y_ref, macc):
        i = pl.program_id(0)

        @pl.when(i == 0)
        def _():
            macc[...] = jnp.zeros_like(macc)

        pm = lax.dot_general(cp_ref[...], hc_ref[...], (((0,), (0,)), ((), ())),
                             preferred_element_type=jnp.float32)
        macc[...] += jnp.sum(pm, axis=0, keepdims=True)

        @pl.when(i == _NG - 1)
        def _():
            o = jnp.maximum(
                jnp.dot(hr_ref[...], ws2_ref[...], preferred_element_type=jnp.float32)
                + jnp.dot(macc[...], wm2_ref[...], preferred_element_type=jnp.float32)
                + b2_ref[...], 0.0)
            lg = jnp.dot(o, wc_ref[...], preferred_element_type=jnp.float32) + bc_ref[...]
            mx = jnp.max(lg, axis=1, keepdims=True)
            ex = jnp.exp(lg - mx)
            y_ref[...] = ex / jnp.sum(ex, axis=1, keepdims=True)

    return pl.pallas_call(
        body,
        grid=(_NG,),
        in_specs=[pl.BlockSpec((_BR, H), lambda i: (i, 0)),
                  pl.BlockSpec((_BR, NS), lambda i: (i, 0)),
                  pl.BlockSpec((1, H), lambda i: (0, 0)),
                  pl.BlockSpec((H, H), lambda i: (0, 0)),
                  pl.BlockSpec((H, H), lambda i: (0, 0)),
                  pl.BlockSpec((1, H), lambda i: (0, 0)),
                  pl.BlockSpec((H, 128), lambda i: (0, 0)),
                  pl.BlockSpec((1, 128), lambda i: (0, 0))],
        out_specs=pl.BlockSpec((1, 128), lambda i: (0, 0)),
        out_shape=jax.ShapeDtypeStruct((1, 128), jnp.float32),
        scratch_shapes=[pltpu.VMEM((1, H), jnp.float32)],
    )(hc, cpart, hrow, ws2, wm2, b2, wcp, bcp)


def _pad2(a, fill):
    a = a.reshape(NS, EPS)
    pad = jnp.full((NS, EPW - EPS), fill, a.dtype)
    return jnp.concatenate([a, pad], axis=1)


def _padc(a, fill):
    return _pad2(a, fill).reshape(NS, NCHUNK, CH)


def kernel(x_d0, x_d1, x_d2, x_d3, x_d4, batch_d0, batch_d1, batch_d2,
           batch_d3, batch_d4, ei0, ei1, ei2, ei3, ei4, ei5, ei6, ei7, ei8,
           ei9, W_in, b_in, Ws1, Wm1, b1, Ws2, Wm2, b2, We, be, Wc, bc, index):
    eis = [ei0, ei1, ei2, ei3, ei4, ei5, ei6, ei7, ei8, ei9]
    x = jnp.concatenate([x_d0, x_d1, x_d2, x_d3], axis=0)
    zacc = jnp.zeros((NP, H), jnp.float32)

    sg = [_padc(eis[j][0] + PAIRS[j][0] * N, 0) for j in range(10)]
    dl = [_padc(eis[j][1], TRASH) for j in range(10)]

    def _acc2msgs(o):
        return jnp.concatenate([o[0, 0, :N], o[0, 1, :N], o[1, 0, :N],
                                o[1, 1, :N]], axis=0)

    h0 = _tc_proj(x, W_in, b_in.reshape(1, H))
    wl4 = [a.reshape(-1) for a in (sg[0], dl[0], sg[1], dl[1],
                                   sg[2], dl[2], sg[3], dl[3])]
    mA = _acc2msgs(_scat4(h0, zacc, *wl4))
    hA = _tc_layer(h0, mA, Ws1, Wm1, b1.reshape(1, H))
    mB = _acc2msgs(_scat4(hA, zacc, *wl4))
    w4 = jnp.concatenate([We[:H, :], We[H:, :]], axis=1)
    hB, d4t = _tc_layer(hA, mB, Ws2, Wm2, b2.reshape(1, H), w4=w4)

    gkey = jax.random.key(12345)
    g0p, g1p = [], []
    for j in range(4):
        u = jax.random.uniform(jax.random.fold_in(gkey, j), (E, 2),
                               minval=1e-6, maxval=1.0 - 1e-6)
        g = -jnp.log(-jnp.log(u))
        g0p.append(_pad2(g[:, 0] + be[0], 0.0))
        g1p.append(_pad2(g[:, 1] + be[1], 0.0))
    g0p = jnp.stack(g0p)
    g1p = jnp.stack(g1p)
    g0p = g0p.reshape(-1)
    g1p = g1p.reshape(-1)
    srcl = jnp.stack([_pad2(eis[j][0], 0) for j in range(4)]).reshape(-1)
    dstl = jnp.stack([_pad2(eis[j][1], N) for j in range(4)]).reshape(-1)
    csrc = jnp.stack([_pad2(eis[j][0], 0) for j in (5, 7, 9)]).reshape(-1)
    cdst = jnp.stack([_pad2(eis[j][1], N) for j in (5, 7, 9)]).reshape(-1)
    idx16 = jnp.full((16,), index, jnp.int32)
    zpad = jnp.zeros((16,), jnp.float32)
    a0t = d4t[:, 0]
    a1t = d4t[:, 1]
    b0t = jnp.concatenate([d4t[:, 2], zpad])
    b1t = jnp.concatenate([d4t[:, 3], zpad])

    ws_o, de_o, cp_o = _attn(a0t, a1t, b0t, b1t, srcl, dstl, g0p, g1p,
                             idx16, csrc, cdst)
    ws_o = ws_o.reshape(4, NS, EPW)
    de_o = de_o.reshape(4, NS, EPW)
    cp_o = cp_o.reshape(NC, NS, CROWS)
    ws = [ws_o[j][:, :EPS].reshape(E) for j in range(4)]

    de = [de_o[j].reshape(NS, NCHUNK, CH) for j in range(4)]
    dg = {j: (de[j] if j < 4 else dl[j]) for j in range(10)}
    wl = []
    for dom in range(4):
        js = _P10_BY_DOM[dom]
        wl.append(jnp.concatenate([sg[j] for j in js], axis=1).reshape(-1))
        wl.append(jnp.concatenate([dg[j] for j in js], axis=1).reshape(-1))
    mC = _acc2msgs(_scat10(h0, zacc, *wl))
    hC = _tc_layer(h0, mC, Ws1, Wm1, b1.reshape(1, H))

    cpart = jnp.concatenate([cp_o[0, :, :2 * N], cp_o[1, :, :2 * N]], axis=1).T
    hrow = lax.dynamic_slice(hC, (N + index, 0), (1, H))
    wcp = jnp.concatenate([Wc, jnp.zeros((H, 120), jnp.float32)], axis=1)
    bcp = jnp.concatenate([bc, jnp.full((120,), -1e30, jnp.float32)]).reshape(1, 128)
    y = _tc_final(hC, cpart, hrow, Ws2, Wm2, b2.reshape(1, H), wcp, bcp)
    y_hat = y[:, :8]
    return (y_hat, ws[0], ws[1], ws[2], ws[3])


# speedup vs baseline: 3.2230x; 3.2230x over previous
import functools

import jax
import jax.numpy as jnp
from jax import lax
from jax.experimental import pallas as pl
from jax.experimental.pallas import tpu as pltpu
from jax.experimental.pallas import tpu_sc as plsc

N = 10000
E = 100000
XD = 200
H = 64
PAIRS = [(0, 0), (1, 1), (2, 2), (3, 3), (1, 0), (0, 1), (1, 2), (2, 1), (1, 3), (3, 1)]

NC, NS, L = 2, 16, 16
CH = 128
EPS = E // NS
NCHUNK = 50
EPW = NCHUNK * CH
TRASH = N
NP = 10112
ZR = NP // NS

_P10_SC0 = ((0, 4), (1, 5, 7))
_P10_SC1 = ((2, 6), (3, 8), (9,))


def _make_scatter(ks0, ks1):
    kmax = max(max(ks0), max(ks1))
    nph = max(len(ks0), len(ks1))
    mesh = plsc.VectorSubcoreMesh(core_axis_name="c", subcore_axis_name="s")

    @functools.partial(
        pl.kernel,
        mesh=mesh,
        out_type=jax.ShapeDtypeStruct((NC, nph, NP, H), jnp.float32),
        scratch_types=[
            pltpu.VMEM((kmax, CH), jnp.int32),
            pltpu.VMEM((kmax, CH), jnp.int32),
            pltpu.VMEM((CH, H), jnp.float32),
            pltpu.VMEM((CH, H), jnp.float32),
            pltpu.VMEM_SHARED((NP, H), jnp.float32),
            pltpu.SemaphoreType.DMA,
            pltpu.SemaphoreType.DMA,
        ],
        compiler_params=pltpu.CompilerParams(use_tc_tiling_on_sc=False,
                                             needs_layout_passes=False),
    )
    def scat(h_hbm, z_hbm, *refs):
        wls, rest = refs[:2 * (len(ks0) + len(ks1))], refs[2 * (len(ks0) + len(ks1)):]
        out, srcv, dstv, rb0, rb1, acc, sm0, sm1 = rest
        cid = lax.axis_index("c")
        sid = lax.axis_index("s")
        stripe = pl.ds(sid * ZR, ZR)

        def run(s_hbm, d_hbm, k):
            pltpu.sync_copy(s_hbm.at[sid], srcv.at[pl.ds(0, k)])
            pltpu.sync_copy(d_hbm.at[sid], dstv.at[pl.ds(0, k)])

            def fire(kk, rb, sm):
                pltpu.async_copy(h_hbm.at[srcv.at[kk]], rb, sm)

            def wait(rb, sm):
                pltpu.make_async_copy(h_hbm.at[srcv.at[0]], rb, sm).wait()

            def scat_add(kk, rb):
                pltpu.sync_copy(rb, acc.at[dstv.at[kk]], add=True)

            fire(0, rb0, sm0)
            fire(1, rb1, sm1)

            def body(i, c):
                kk = 2 * i
                wait(rb0, sm0)
                scat_add(kk, rb0)
                fire(kk + 2, rb0, sm0)
                wait(rb1, sm1)
                scat_add(kk + 1, rb1)
                fire(kk + 3, rb1, sm1)
                return c

            lax.fori_loop(0, k // 2 - 1, body, 0)
            wait(rb0, sm0)
            scat_add(k - 2, rb0)
            wait(rb1, sm1)
            scat_add(k - 1, rb1)

        def phase(t, s_hbm, d_hbm, k):
            pltpu.sync_copy(z_hbm.at[stripe], acc.at[stripe])
            plsc.subcore_barrier()
            run(s_hbm, d_hbm, k)
            plsc.subcore_barrier()
            pltpu.sync_copy(acc.at[stripe], out.at[cid, t, stripe])

        @pl.when(cid == 0)
        def _():
            for t, k in enumerate(ks0):
                phase(t, wls[2 * t], wls[2 * t + 1], k)

        @pl.when(cid == 1)
        def _():
            o = 2 * len(ks0)
            for t, k in enumerate(ks1):
                phase(t, wls[o + 2 * t], wls[o + 2 * t + 1], k)

    return scat


_scat4 = _make_scatter((NCHUNK, NCHUNK), (NCHUNK, NCHUNK))
_scat10 = _make_scatter(tuple(len(p) * NCHUNK for p in _P10_SC0),
                        tuple(len(p) * NCHUNK for p in _P10_SC1))


CROWS = 2 * N + 16
mesh_a = plsc.VectorSubcoreMesh(core_axis_name="c", subcore_axis_name="s")


@functools.partial(
    pl.kernel,
    mesh=mesh_a,
    out_type=[
        jax.ShapeDtypeStruct((4 * NS * EPW,), jnp.float32),
        jax.ShapeDtypeStruct((4 * NS * EPW,), jnp.int32),
        jax.ShapeDtypeStruct((NC * NS * CROWS,), jnp.float32)
    ],
    scratch_types=[
        pltpu.VMEM((N,), jnp.float32),
        pltpu.VMEM((N,), jnp.float32),
        pltpu.VMEM((N + 16,), jnp.float32),
        pltpu.VMEM((N + 16,), jnp.float32),
        pltpu.VMEM((EPW,), jnp.int32),
        pltpu.VMEM((EPW,), jnp.int32),
        pltpu.VMEM((EPW,), jnp.float32),
        pltpu.VMEM((EPW,), jnp.float32),
        pltpu.VMEM((EPW,), jnp.float32),
        pltpu.VMEM((EPW,), jnp.int32),
        pltpu.VMEM((16,), jnp.int32),
        pltpu.VMEM((CROWS,), jnp.float32),
    ],
    compiler_params=pltpu.CompilerParams(use_tc_tiling_on_sc=False,
                                         needs_layout_passes=False),
)
def _attn(a0t, a1t, b0t, b1t, srcl, dstl, g0, g1, idx16, csrc, cdst,
          ws_o, de_o, cp_o,
          a0s, a1s, b0s, b1s, srcf, dstf, g0f, g1f, wsb, deb, idxv_v, cacc):
    cid = lax.axis_index("c")
    sid = lax.axis_index("s")
    pltpu.sync_copy(idx16, idxv_v)
    idxv = idxv_v[...]
    nv = jnp.full((16,), N, jnp.int32)

    def zero(i, c):
        cacc[pl.ds(i * 16, 16)] = jnp.zeros((16,), jnp.float32)
        return c

    lax.fori_loop(0, CROWS // 16, zero, 0)

    def wspair(p, with_c, coff):
        pltpu.sync_copy(a0t.at[pl.ds(p * N, N)], a0s)
        pltpu.sync_copy(a1t.at[pl.ds(p * N, N)], a1s)
        pltpu.sync_copy(b0t.at[pl.ds(p * N, N + 16)], b0s)
        pltpu.sync_copy(b1t.at[pl.ds(p * N, N + 16)], b1s)
        seg = pl.ds((p * NS + sid) * EPW, EPW)
        pltpu.sync_copy(srcl.at[seg], srcf)
        pltpu.sync_copy(dstl.at[seg], dstf)
        pltpu.sync_copy(g0.at[seg], g0f)
        pltpu.sync_copy(g1.at[seg], g1f)
        cofv = jnp.full((16,), coff, jnp.int32)

        def body(i, c):
            o = i * 16
            vs = srcf[pl.ds(o, 16)]
            vd = dstf[pl.ds(o, 16)]
            m0 = (plsc.load_gather(a0s, [vs]) + plsc.load_gather(b0s, [vd])) \
                + g0f[pl.ds(o, 16)]
            m1 = (plsc.load_gather(a1s, [vs]) + plsc.load_gather(b1s, [vd])) \
                + g1f[pl.ds(o, 16)]
            w = m1 > m0
            wsb[pl.ds(o, 16)] = jnp.where(w, 1.0, 0.0).astype(jnp.float32)
            deb[pl.ds(o, 16)] = jnp.where(w, vd, nv)
            if with_c:
                coeff = jnp.where(w & (vd == idxv), 1.0, 0.0).astype(jnp.float32)
                plsc.addupdate_scatter(cacc, [vs + cofv], coeff)
            return c

        lax.fori_loop(0, EPW // 16, body, 0)
        pltpu.sync_copy(wsb, ws_o.at[seg])
        pltpu.sync_copy(deb, de_o.at[seg])

    def cpair(k, coff):
        seg = pl.ds((k * NS + sid) * EPW, EPW)
        pltpu.sync_copy(csrc.at[seg], srcf)
        pltpu.sync_copy(cdst.at[seg], dstf)
        cofv = jnp.full((16,), coff, jnp.int32)

        def body(i, c):
            o = i * 16
            vs = srcf[pl.ds(o, 16)]
            vd = dstf[pl.ds(o, 16)]
            coeff = jnp.where(vd == idxv, 1.0, 0.0).astype(jnp.float32)
            plsc.addupdate_scatter(cacc, [vs + cofv], coeff)
            return c

        lax.fori_loop(0, EPW // 16, body, 0)

    @pl.when(cid == 0)
    def _():
        wspair(0, False, 0)
        wspair(1, True, N)
        cpair(0, 0)

    @pl.when(cid == 1)
    def _():
        wspair(2, False, 0)
        wspair(3, False, 0)
        cpair(1, 0)
        cpair(2, N)

    pltpu.sync_copy(cacc, cp_o.at[pl.ds((cid * NS + sid) * CROWS, CROWS)])


M4 = 4 * N
_BM = 2000


def _tc_proj(x, w, b):
    def body(x_ref, w_ref, b_ref, o_ref):
        o_ref[...] = jnp.maximum(
            jnp.dot(x_ref[...], w_ref[...], preferred_element_type=jnp.float32)
            + b_ref[...], 0.0)

    return pl.pallas_call(
        body,
        grid=(M4 // _BM,),
        in_specs=[pl.BlockSpec((_BM, XD), lambda i: (i, 0)),
                  pl.BlockSpec((XD, H), lambda i: (0, 0)),
                  pl.BlockSpec((1, H), lambda i: (0, 0))],
        out_specs=pl.BlockSpec((_BM, H), lambda i: (i, 0)),
        out_shape=jax.ShapeDtypeStruct((M4, H), jnp.float32),
    )(x, w, b)


def _tc_layer(h, m, ws, wm, b, w4=None, m2=None):
    if w4 is None and m2 is None:
        def body(h_ref, m_ref, ws_ref, wm_ref, b_ref, o_ref):
            o_ref[...] = jnp.maximum(
                jnp.dot(h_ref[...], ws_ref[...], preferred_element_type=jnp.float32)
                + jnp.dot(m_ref[...], wm_ref[...], preferred_element_type=jnp.float32)
                + b_ref[...], 0.0)

        return pl.pallas_call(
            body,
            grid=(M4 // _BM,),
            in_specs=[pl.BlockSpec((_BM, H), lambda i: (i, 0)),
                      pl.BlockSpec((_BM, H), lambda i: (i, 0)),
                      pl.BlockSpec((H, H), lambda i: (0, 0)),
                      pl.BlockSpec((H, H), lambda i: (0, 0)),
                      pl.BlockSpec((1, H), lambda i: (0, 0))],
            out_specs=pl.BlockSpec((_BM, H), lambda i: (i, 0)),
            out_shape=jax.ShapeDtypeStruct((M4, H), jnp.float32),
        )(h, m, ws, wm, b)

    if m2 is not None:
        lo, hi = N // _BM, 2 * N // _BM

        def body(h_ref, m_ref, m2_ref, ws_ref, wm_ref, b_ref, o_ref):
            i = pl.program_id(0)
            f = ((i >= lo) & (i < hi)).astype(jnp.float32)
            mm = m_ref[...] + f * m2_ref[...]
            o_ref[...] = jnp.maximum(
                jnp.dot(h_ref[...], ws_ref[...], preferred_element_type=jnp.float32)
                + jnp.dot(mm, wm_ref[...], preferred_element_type=jnp.float32)
                + b_ref[...], 0.0)

        return pl.pallas_call(
            body,
            grid=(M4 // _BM,),
            in_specs=[pl.BlockSpec((_BM, H), lambda i: (i, 0)),
                      pl.BlockSpec((_BM, H), lambda i: (i, 0)),
                      pl.BlockSpec((_BM, H),
                                   lambda i: (jnp.clip(i - lo, 0, hi - lo - 1), 0)),
                      pl.BlockSpec((H, H), lambda i: (0, 0)),
                      pl.BlockSpec((H, H), lambda i: (0, 0)),
                      pl.BlockSpec((1, H), lambda i: (0, 0))],
            out_specs=pl.BlockSpec((_BM, H), lambda i: (i, 0)),
            out_shape=jax.ShapeDtypeStruct((M4, H), jnp.float32),
        )(h, m, m2, ws, wm, b)

    def body(h_ref, m_ref, ws_ref, wm_ref, b_ref, w4_ref, o_ref, t_ref):
        o = jnp.maximum(
            jnp.dot(h_ref[...], ws_ref[...], preferred_element_type=jnp.float32)
            + jnp.dot(m_ref[...], wm_ref[...], preferred_element_type=jnp.float32)
            + b_ref[...], 0.0)
        o_ref[...] = o
        t_ref[...] = jnp.dot(o, w4_ref[...], preferred_element_type=jnp.float32)

    return pl.pallas_call(
        body,
        grid=(M4 // _BM,),
        in_specs=[pl.BlockSpec((_BM, H), lambda i: (i, 0)),
                  pl.BlockSpec((_BM, H), lambda i: (i, 0)),
                  pl.BlockSpec((H, H), lambda i: (0, 0)),
                  pl.BlockSpec((H, H), lambda i: (0, 0)),
                  pl.BlockSpec((1, H), lambda i: (0, 0)),
                  pl.BlockSpec((H, 4), lambda i: (0, 0))],
        out_specs=[pl.BlockSpec((_BM, H), lambda i: (i, 0)),
                   pl.BlockSpec((_BM, 4), lambda i: (i, 0))],
        out_shape=[jax.ShapeDtypeStruct((M4, H), jnp.float32),
                   jax.ShapeDtypeStruct((M4, 4), jnp.float32)],
    )(h, m, ws, wm, b, w4)


_BR = 2000
_NG = M4 // _BR


def _tc_final(hc, cpart, hrow, ws2, wm2, b2, wcp, bcp):
    def body(hc_ref, cp_ref, hr_ref, ws2_ref, wm2_ref, b2_ref, wc_ref,
             bc_ref, y_ref, macc):
        i = pl.program_id(0)

        @pl.when(i == 0)
        def _():
            macc[...] = jnp.zeros_like(macc)

        pm = lax.dot_general(cp_ref[...], hc_ref[...], (((0,), (0,)), ((), ())),
                             preferred_element_type=jnp.float32)
        macc[...] += jnp.sum(pm, axis=0, keepdims=True)

        @pl.when(i == _NG - 1)
        def _():
            o = jnp.maximum(
                jnp.dot(hr_ref[...], ws2_ref[...], preferred_element_type=jnp.float32)
                + jnp.dot(macc[...], wm2_ref[...], preferred_element_type=jnp.float32)
                + b2_ref[...], 0.0)
            lg = jnp.dot(o, wc_ref[...], preferred_element_type=jnp.float32) + bc_ref[...]
            mx = jnp.max(lg, axis=1, keepdims=True)
            ex = jnp.exp(lg - mx)
            y_ref[...] = ex / jnp.sum(ex, axis=1, keepdims=True)

    return pl.pallas_call(
        body,
        grid=(_NG,),
        in_specs=[pl.BlockSpec((_BR, H), lambda i: (i, 0)),
                  pl.BlockSpec((_BR, NS), lambda i: (i, 0)),
                  pl.BlockSpec((1, H), lambda i: (0, 0)),
                  pl.BlockSpec((H, H), lambda i: (0, 0)),
                  pl.BlockSpec((H, H), lambda i: (0, 0)),
                  pl.BlockSpec((1, H), lambda i: (0, 0)),
                  pl.BlockSpec((H, 128), lambda i: (0, 0)),
                  pl.BlockSpec((1, 128), lambda i: (0, 0))],
        out_specs=pl.BlockSpec((1, 128), lambda i: (0, 0)),
        out_shape=jax.ShapeDtypeStruct((1, 128), jnp.float32),
        scratch_shapes=[pltpu.VMEM((1, H), jnp.float32)],
    )(hc, cpart, hrow, ws2, wm2, b2, wcp, bcp)


def _pad2(a, fill):
    a = a.reshape(NS, EPS)
    pad = jnp.full((NS, EPW - EPS), fill, a.dtype)
    return jnp.concatenate([a, pad], axis=1)


def _padc(a, fill):
    return _pad2(a, fill).reshape(NS, NCHUNK, CH)


def kernel(x_d0, x_d1, x_d2, x_d3, x_d4, batch_d0, batch_d1, batch_d2,
           batch_d3, batch_d4, ei0, ei1, ei2, ei3, ei4, ei5, ei6, ei7, ei8,
           ei9, W_in, b_in, Ws1, Wm1, b1, Ws2, Wm2, b2, We, be, Wc, bc, index):
    eis = [ei0, ei1, ei2, ei3, ei4, ei5, ei6, ei7, ei8, ei9]
    x = jnp.concatenate([x_d0, x_d1, x_d2, x_d3], axis=0)
    zacc = jnp.zeros((NP, H), jnp.float32)

    sg = [_padc(eis[j][0] + PAIRS[j][0] * N, 0) for j in range(10)]
    dl = [_padc(eis[j][1], TRASH) for j in range(10)]

    def _acc2msgs(o):
        return jnp.concatenate([o[0, 0, :N], o[0, 1, :N], o[1, 0, :N],
                                o[1, 1, :N]], axis=0)

    h0 = _tc_proj(x, W_in, b_in.reshape(1, H))
    wl4 = (sg[0], dl[0], sg[1], dl[1], sg[2], dl[2], sg[3], dl[3])
    mA = _acc2msgs(_scat4(h0, zacc, *wl4))
    hA = _tc_layer(h0, mA, Ws1, Wm1, b1.reshape(1, H))
    mB = _acc2msgs(_scat4(hA, zacc, *wl4))
    w4 = jnp.concatenate([We[:H, :], We[H:, :]], axis=1)
    hB, d4t = _tc_layer(hA, mB, Ws2, Wm2, b2.reshape(1, H), w4=w4)

    gkey = jax.random.key(12345)
    g0p, g1p = [], []
    for j in range(4):
        u = jax.random.uniform(jax.random.fold_in(gkey, j), (E, 2),
                               minval=1e-6, maxval=1.0 - 1e-6)
        g = -jnp.log(-jnp.log(u))
        g0p.append(_pad2(g[:, 0] + be[0], 0.0))
        g1p.append(_pad2(g[:, 1] + be[1], 0.0))
    g0p = jnp.stack(g0p)
    g1p = jnp.stack(g1p)
    g0p = g0p.reshape(-1)
    g1p = g1p.reshape(-1)
    srcl = jnp.stack([_pad2(eis[j][0], 0) for j in range(4)]).reshape(-1)
    dstl = jnp.stack([_pad2(eis[j][1], N) for j in range(4)]).reshape(-1)
    csrc = jnp.stack([_pad2(eis[j][0], 0) for j in (5, 7, 9)]).reshape(-1)
    cdst = jnp.stack([_pad2(eis[j][1], N) for j in (5, 7, 9)]).reshape(-1)
    idx16 = jnp.full((16,), index, jnp.int32)
    zpad = jnp.zeros((16,), jnp.float32)
    a0t = d4t[:, 0]
    a1t = d4t[:, 1]
    b0t = jnp.concatenate([d4t[:, 2], zpad])
    b1t = jnp.concatenate([d4t[:, 3], zpad])

    ws_o, de_o, cp_o = _attn(a0t, a1t, b0t, b1t, srcl, dstl, g0p, g1p,
                             idx16, csrc, cdst)
    ws_o = ws_o.reshape(4, NS, EPW)
    de_o = de_o.reshape(4, NS, EPW)
    cp_o = cp_o.reshape(NC, NS, CROWS)
    ws = [ws_o[j][:, :EPS].reshape(E) for j in range(4)]

    de = [de_o[j].reshape(NS, NCHUNK, CH) for j in range(4)]
    dg = {j: (de[j] if j < 4 else dl[j]) for j in range(10)}
    wl = []
    for phases in (_P10_SC0, _P10_SC1):
        for js in phases:
            wl.append(jnp.concatenate([sg[j] for j in js], axis=1))
            wl.append(jnp.concatenate([dg[j] for j in js], axis=1))
    oc = _scat10(h0, zacc, *wl)
    mC = jnp.concatenate([oc[0, 0, :N], oc[0, 1, :N], oc[1, 0, :N],
                          oc[1, 1, :N]], axis=0)
    hC = _tc_layer(h0, mC, Ws1, Wm1, b1.reshape(1, H), m2=oc[1, 2, :N])

    cpart = jnp.concatenate([cp_o[0, :, :2 * N], cp_o[1, :, :2 * N]], axis=1).T
    hrow = lax.dynamic_slice(hC, (N + index, 0), (1, H))
    wcp = jnp.concatenate([Wc, jnp.zeros((H, 120), jnp.float32)], axis=1)
    bcp = jnp.concatenate([bc, jnp.full((120,), -1e30, jnp.float32)]).reshape(1, 128)
    y = _tc_final(hC, cpart, hrow, Ws2, Wm2, b2.reshape(1, H), wcp, bcp)
    y_hat = y[:, :8]
    return (y_hat, ws[0], ws[1], ws[2], ws[3])
